# baseline (speedup 1.0000x reference)
import functools

import jax
import jax.numpy as jnp
from jax import lax
from jax.experimental import pallas as pl
from jax.experimental.pallas import tpu as pltpu
from jax.experimental.pallas import tpu_sc as plsc

_PROMPT_LAYER_INDICES = (0,)
_NC, _NS, _L = 2, 16, 16
_NW = _NC * _NS


@functools.cache
def _build_sc_gather(B, T, E, NT, P, zero_prefix):
    PR = B * P
    assert PR % _NW == 0
    pr_w = PR // _NW
    assert pr_w == _L
    w_per_b = _NW // B
    assert P == w_per_b * pr_w

    mesh = plsc.VectorSubcoreMesh(core_axis_name="c", subcore_axis_name="s")

    def body(tab_hbm, pidx_hbm, out_hbm, idx_v, pbuf, sg, sp):
        wid = lax.axis_index("s") * _NC + lax.axis_index("c")
        w_b = wid // w_per_b
        w_c = wid % w_per_b
        if zero_prefix:
            def zcol(j, carry):
                for r in range(_L):
                    pbuf[r, pl.ds(j * _L, _L)] = jnp.zeros((_L,), jnp.float32)
                return carry
            lax.fori_loop(0, E // _L, zcol, 0)
        else:
            pltpu.sync_copy(pidx_hbm.at[pl.ds(wid * pr_w, pr_w)], idx_v)
            gather = pltpu.make_async_copy(tab_hbm.at[idx_v], pbuf, sg)
            gather.start()
            gather.wait()
        orow0 = w_b * (P + T) + w_c * pr_w
        store = pltpu.make_async_copy(
            pbuf, out_hbm.at[pl.ds(orow0, pr_w)], sp)
        store.start()
        store.wait()

    return pl.kernel(
        body,
        out_type=jax.ShapeDtypeStruct((B * (P + T), E), jnp.float32),
        mesh=mesh,
        scratch_types=[
            pltpu.VMEM((_L,), jnp.int32),
            pltpu.VMEM((pr_w, E), jnp.float32),
            pltpu.SemaphoreType.DMA,
            pltpu.SemaphoreType.DMA,
        ],
    )


@functools.cache
def _build_tc_assemble(B, T, E, P):
    CR = 1024
    NBUF, LA = 6, 3
    assert T % CR == 0
    chunks = []
    for b in range(B):
        for j in range(T // CR):
            chunks.append((b * T + j * CR, b * (P + T) + P + j * CR))
    n = len(chunks)

    def body(al_ref, in_ref, o_ref, bufs, isem, osem):
        del al_ref

        def in_copy(k):
            sr, _ = chunks[k]
            return pltpu.make_async_copy(
                in_ref.at[pl.ds(sr, CR)], bufs.at[k % NBUF], isem.at[k % NBUF])

        def out_copy(k):
            _, orow = chunks[k]
            return pltpu.make_async_copy(
                bufs.at[k % NBUF], o_ref.at[pl.ds(orow, CR)], osem.at[k % NBUF])

        for j in range(LA):
            in_copy(j).start()
        for k in range(n):
            in_copy(k).wait()
            out_copy(k).start()
            if k + LA < n:
                if k >= NBUF - LA:
                    out_copy(k + LA - NBUF).wait()
                in_copy(k + LA).start()
        for k in range(n - NBUF, n):
            out_copy(k).wait()

    return pl.pallas_call(
        body,
        in_specs=[
            pl.BlockSpec(memory_space=pl.ANY),
            pl.BlockSpec(memory_space=pl.ANY),
        ],
        out_specs=pl.BlockSpec(memory_space=pl.ANY),
        out_shape=jax.ShapeDtypeStruct((B * (P + T), E), jnp.float32),
        input_output_aliases={0: 0},
        scratch_shapes=[
            pltpu.VMEM((NBUF, CR, E), jnp.float32),
            pltpu.SemaphoreType.DMA((NBUF,)),
            pltpu.SemaphoreType.DMA((NBUF,)),
        ],
    )


def kernel(input_embedding, layer_idx, task_ids, prompt_table):
    B, T, E = input_embedding.shape
    NT, P, _ = prompt_table.shape
    if P == 0:
        return input_embedding
    active = jnp.any(
        jnp.asarray(_PROMPT_LAYER_INDICES, jnp.int32)
        == jnp.asarray(layer_idx, jnp.int32))
    in_rows = input_embedding.reshape(B * T, E)
    tab_rows = prompt_table.reshape(NT * P, E)
    prow_idx = (task_ids.astype(jnp.int32)[:, None] * P
                + jnp.arange(P, dtype=jnp.int32)[None, :]).reshape(B * P)
    pre = lax.cond(
        active,
        lambda a, b: _build_sc_gather(B, T, E, NT, P, False)(a, b),
        lambda a, b: _build_sc_gather(B, T, E, NT, P, True)(a, b),
        tab_rows, prow_idx)
    out = _build_tc_assemble(B, T, E, P)(pre, in_rows)
    return out.reshape(B, P + T, E)


# speedup vs baseline: 1.0029x; 1.0029x over previous
import functools

import jax
import jax.numpy as jnp
from jax import lax
from jax.experimental import pallas as pl
from jax.experimental.pallas import tpu as pltpu
from jax.experimental.pallas import tpu_sc as plsc

_PROMPT_LAYER_INDICES = (0,)
_NC, _NS, _L = 2, 16, 16
_NW = _NC * _NS


@functools.cache
def _build_sc_gather(B, T, E, NT, P, zero_prefix):
    PR = B * P
    assert PR % _NW == 0
    pr_w = PR // _NW
    assert pr_w == _L
    w_per_b = _NW // B
    assert P == w_per_b * pr_w

    mesh = plsc.VectorSubcoreMesh(core_axis_name="c", subcore_axis_name="s")

    def body(tab_hbm, pidx_hbm, out_hbm, idx_v, pbuf, sg, sp):
        wid = lax.axis_index("s") * _NC + lax.axis_index("c")
        w_b = wid // w_per_b
        w_c = wid % w_per_b
        if zero_prefix:
            def zcol(j, carry):
                for r in range(_L):
                    pbuf[r, pl.ds(j * _L, _L)] = jnp.zeros((_L,), jnp.float32)
                return carry
            lax.fori_loop(0, E // _L, zcol, 0)
        else:
            pltpu.sync_copy(pidx_hbm.at[pl.ds(wid * pr_w, pr_w)], idx_v)
            gather = pltpu.make_async_copy(tab_hbm.at[idx_v], pbuf, sg)
            gather.start()
            gather.wait()
        orow0 = w_b * (P + T) + w_c * pr_w
        store = pltpu.make_async_copy(
            pbuf, out_hbm.at[pl.ds(orow0, pr_w)], sp)
        store.start()
        store.wait()

    return pl.kernel(
        body,
        out_type=jax.ShapeDtypeStruct((B * (P + T), E), jnp.float32),
        mesh=mesh,
        scratch_types=[
            pltpu.VMEM((_L,), jnp.int32),
            pltpu.VMEM((pr_w, E), jnp.float32),
            pltpu.SemaphoreType.DMA,
            pltpu.SemaphoreType.DMA,
        ],
    )


@functools.cache
def _build_tc_assemble(B, T, E, P):
    CR = 1024
    NBUF, LA = 4, 2
    assert T % CR == 0
    chunks = []
    for b in range(B):
        for j in range(T // CR):
            chunks.append((b * T + j * CR, b * (P + T) + P + j * CR))
    n = len(chunks)

    def body(al_ref, in_ref, o_ref, bufs, isem, osem):
        del al_ref

        def in_copy(k):
            sr, _ = chunks[k]
            return pltpu.make_async_copy(
                in_ref.at[pl.ds(sr, CR)], bufs.at[k % NBUF], isem.at[k % NBUF])

        def out_copy(k):
            _, orow = chunks[k]
            return pltpu.make_async_copy(
                bufs.at[k % NBUF], o_ref.at[pl.ds(orow, CR)], osem.at[k % NBUF])

        for j in range(LA):
            in_copy(j).start()
        for k in range(n):
            in_copy(k).wait()
            out_copy(k).start()
            if k + LA < n:
                if k >= NBUF - LA:
                    out_copy(k + LA - NBUF).wait()
                in_copy(k + LA).start()
        for k in range(n - NBUF, n):
            out_copy(k).wait()

    return pl.pallas_call(
        body,
        in_specs=[
            pl.BlockSpec(memory_space=pl.ANY),
            pl.BlockSpec(memory_space=pl.ANY),
        ],
        out_specs=pl.BlockSpec(memory_space=pl.ANY),
        out_shape=jax.ShapeDtypeStruct((B * (P + T), E), jnp.float32),
        input_output_aliases={0: 0},
        scratch_shapes=[
            pltpu.VMEM((NBUF, CR, E), jnp.float32),
            pltpu.SemaphoreType.DMA((NBUF,)),
            pltpu.SemaphoreType.DMA((NBUF,)),
        ],
    )


def kernel(input_embedding, layer_idx, task_ids, prompt_table):
    B, T, E = input_embedding.shape
    NT, P, _ = prompt_table.shape
    if P == 0:
        return input_embedding
    active = jnp.any(
        jnp.asarray(_PROMPT_LAYER_INDICES, jnp.int32)
        == jnp.asarray(layer_idx, jnp.int32))
    in_rows = input_embedding.reshape(B * T, E)
    tab_rows = prompt_table.reshape(NT * P, E)
    prow_idx = (task_ids.astype(jnp.int32)[:, None] * P
                + jnp.arange(P, dtype=jnp.int32)[None, :]).reshape(B * P)
    pre = lax.cond(
        active,
        lambda a, b: _build_sc_gather(B, T, E, NT, P, False)(a, b),
        lambda a, b: _build_sc_gather(B, T, E, NT, P, True)(a, b),
        tab_rows, prow_idx)
    out = _build_tc_assemble(B, T, E, P)(pre, in_rows)
    return out.reshape(B, P + T, E)
